# Initial kernel scaffold; baseline (speedup 1.0000x reference)
import jax
import jax.numpy as jnp
from jax.experimental import pallas as pl


def kernel(inputs, table, training):
    raise NotImplementedError("write your pallas kernel here")


# speedup vs baseline: 2.0187x; 2.0187x over previous
import functools

import jax
import jax.numpy as jnp
import numpy as np
from jax import lax
from jax.experimental import pallas as pl
from jax.experimental.pallas import tpu as pltpu
from jax.experimental.pallas import tpu_sc as plsc

_D = 128
_L = 200
_SCALE = float(np.sqrt(_D))
_LANES = 16

_NC = 2
_NS = 16
_NW = _NC * _NS

_W = 128


def _make_pe(seq_len, d):
    pos = np.arange(seq_len)[:, None].astype(np.float64)
    i = np.arange(d)[None, :].astype(np.float64)
    angle = pos / np.power(10000.0, 2.0 * np.floor(i / 2.0) / d)
    pe = np.zeros((seq_len, d), dtype=np.float64)
    pe[:, 0::2] = np.sin(angle[:, 0::2])
    pe[:, 1::2] = np.cos(angle[:, 1::2])
    return pe.astype(np.float32)


_PE = _make_pe(_L, _D)


@functools.partial(jax.jit, static_argnames=("n",))
def _sc_gather_pe(table, flat_idx, pe, n):
    per_w = n // _NW
    n_win = per_w // _W
    mesh = plsc.VectorSubcoreMesh(core_axis_name="c", subcore_axis_name="s")

    @functools.partial(
        pl.kernel,
        out_type=jax.ShapeDtypeStruct((n, _D), jnp.float32),
        mesh=mesh,
        scratch_types=[
            pltpu.VMEM((_W,), jnp.int32),
            pltpu.VMEM((_W, _D), jnp.float32),
            pltpu.VMEM((_L, _D), jnp.float32),
            pltpu.SemaphoreType.DMA,
        ],
    )
    def k(table_hbm, idx_hbm, pe_hbm, out_hbm, idx_v, rows_v, pe_v, sem):
        wid = lax.axis_index("s") * _NC + lax.axis_index("c")
        base0 = wid * per_w
        pltpu.sync_copy(pe_hbm, pe_v)

        @pl.loop(0, n_win)
        def _win(w):
            local = w * _W
            base = base0 + local
            pltpu.sync_copy(idx_hbm.at[pl.ds(base, _W)], idx_v)
            pltpu.async_copy(table_hbm.at[idx_v], rows_v, sem).wait()

            @pl.loop(0, _W)
            def _row(r):
                p = lax.rem(local + r, _L)
                for c in range(_D // _LANES):
                    sl = pl.ds(c * _LANES, _LANES)
                    rows_v[r, sl] = rows_v[r, sl] * _SCALE + pe_v[p, sl]

            pltpu.sync_copy(rows_v, out_hbm.at[pl.ds(base, _W)])

    return k(table, flat_idx, pe)


def kernel(inputs, table, training):
    b, l = inputs.shape
    n = b * l
    flat_idx = inputs.reshape(n)
    out = _sc_gather_pe(table, flat_idx, _PE, n)
    return out.reshape(b, l, _D)


# speedup vs baseline: 3.0736x; 1.5226x over previous
import functools

import jax
import jax.numpy as jnp
import numpy as np
from jax import lax
from jax.experimental import pallas as pl
from jax.experimental.pallas import tpu as pltpu
from jax.experimental.pallas import tpu_sc as plsc

_D = 128
_L = 200
_SCALE = float(np.sqrt(_D))
_LANES = 16

_NC = 2
_NS = 16
_NW = _NC * _NS

_W = 128
_NBUF = 2


def _make_pe(seq_len, d):
    pos = np.arange(seq_len)[:, None].astype(np.float64)
    i = np.arange(d)[None, :].astype(np.float64)
    angle = pos / np.power(10000.0, 2.0 * np.floor(i / 2.0) / d)
    pe = np.zeros((seq_len, d), dtype=np.float64)
    pe[:, 0::2] = np.sin(angle[:, 0::2])
    pe[:, 1::2] = np.cos(angle[:, 1::2])
    return pe.astype(np.float32)


_PE = _make_pe(_L, _D)


@functools.partial(jax.jit, static_argnames=("n",))
def _sc_gather_pe(table, idx2d, pe, n):
    per_w = n // _NW
    n_win = per_w // _W
    idx_rows = per_w // _W
    mesh = plsc.VectorSubcoreMesh(core_axis_name="c", subcore_axis_name="s")

    @functools.partial(
        pl.kernel,
        out_type=jax.ShapeDtypeStruct((n, _D), jnp.float32),
        mesh=mesh,
        scratch_types=[
            pltpu.VMEM((idx_rows, _W), jnp.int32),
            pltpu.VMEM((_L, _D), jnp.float32),
            pltpu.VMEM((_W, _D), jnp.float32),
            pltpu.VMEM((_W, _D), jnp.float32),
            pltpu.VMEM((_W, _D), jnp.float32),
            pltpu.VMEM((_W, _D), jnp.float32),
            pltpu.SemaphoreType.DMA,
            pltpu.SemaphoreType.DMA,
            pltpu.SemaphoreType.DMA,
            pltpu.SemaphoreType.DMA,
        ],
    )
    def k(table_hbm, idx_hbm, pe_hbm, out_hbm,
          idx_v, pe_v, g0, g1, o0, o1, gs0, gs1, os0, os1):
        grows = [g0, g1]
        orows = [o0, o1]
        gsem = [gs0, gs1]
        osem = [os0, os1]
        wid = lax.axis_index("s") * _NC + lax.axis_index("c")
        base0 = wid * per_w

        pltpu.sync_copy(idx_hbm.at[pl.ds(wid * idx_rows, idx_rows)], idx_v)
        pltpu.sync_copy(pe_hbm, pe_v)

        def start_gather(b, w):
            pltpu.async_copy(table_hbm.at[idx_v.at[w]], grows[b], gsem[b])

        def wait_gather(b, w):
            pltpu.make_async_copy(table_hbm.at[idx_v.at[w]], grows[b],
                                  gsem[b]).wait()

        def start_out(b, w):
            pltpu.async_copy(orows[b], out_hbm.at[pl.ds(base0 + w * _W, _W)],
                             osem[b])

        def wait_out(b, w):
            pltpu.make_async_copy(orows[b],
                                  out_hbm.at[pl.ds(base0 + w * _W, _W)],
                                  osem[b]).wait()

        for b in range(_NBUF):
            start_gather(b, b)

        @pl.loop(0, n_win, step=_NBUF)
        def _win(w0):
            for b in range(_NBUF):
                w = w0 + b
                wait_gather(b, w)

                @pl.when(w >= _NBUF)
                def _():
                    wait_out(b, w - _NBUF)

                g = grows[b]
                o = orows[b]

                @pl.loop(0, _W)
                def _row(r):
                    p = lax.rem(w * _W + r, _L)
                    for c in range(_D // _LANES):
                        sl = pl.ds(c * _LANES, _LANES)
                        o[r, sl] = g[r, sl] * _SCALE + pe_v[p, sl]

                @pl.when(w + _NBUF < n_win)
                def _():
                    start_gather(b, w + _NBUF)

                start_out(b, w)

        for b in range(_NBUF):
            wait_out(b, n_win - _NBUF + b)

    return k(table, idx2d, pe)


def kernel(inputs, table, training):
    b, l = inputs.shape
    n = b * l
    idx2d = inputs.reshape(n // _W, _W)
    out = _sc_gather_pe(table, idx2d, _PE, n)
    return out.reshape(b, l, _D)
